# baseline (speedup 1.0000x reference)
import functools

import jax
import jax.numpy as jnp
from jax import lax
from jax.experimental import pallas as pl
from jax.experimental.pallas import tpu as pltpu
from jax.experimental.pallas import tpu_sc as plsc

B = 16384
D = 64
E = 1024
NC = 2
NS = 16
NW = NC * NS
BPW = B // NW
CH = 128
NCH = BPW // CH
IDX_COLS = 128
ROWS_PER_W = NCH

_mesh = plsc.VectorSubcoreMesh(core_axis_name="c", subcore_axis_name="s")


def _wid():
    return lax.axis_index("s") * NC + lax.axis_index("c")


def _sc_gather3_body(uid_h, pos_h, neg_h, item_h, user_h,
                     ue_o, po_o, no_o,
                     uidx, pidx, nidx, urows, prows, nrows, sem, semw):
    wid = _wid()
    base = wid * BPW
    row0 = wid * ROWS_PER_W
    pltpu.sync_copy(uid_h.at[pl.ds(row0, ROWS_PER_W)], uidx)
    pltpu.sync_copy(pos_h.at[pl.ds(row0, ROWS_PER_W)], pidx)
    pltpu.sync_copy(neg_h.at[pl.ds(row0, ROWS_PER_W)], nidx)
    copies = []
    for j in range(NCH):
        sl = pl.ds(j * CH, CH)
        copies.append(pltpu.async_copy(user_h.at[uidx.at[j]], urows.at[sl], sem))
        copies.append(pltpu.async_copy(item_h.at[pidx.at[j]], prows.at[sl], sem))
        copies.append(pltpu.async_copy(item_h.at[nidx.at[j]], nrows.at[sl], sem))
    for c in copies:
        c.wait()
    out_sl = pl.ds(base, BPW)
    w0 = pltpu.async_copy(urows, ue_o.at[out_sl], semw)
    w1 = pltpu.async_copy(prows, po_o.at[out_sl], semw)
    w2 = pltpu.async_copy(nrows, no_o.at[out_sl], semw)
    w0.wait(); w1.wait(); w2.wait()


_sc_gather3 = pl.kernel(
    _sc_gather3_body,
    out_type=[jax.ShapeDtypeStruct((B, D), jnp.float32)] * 3,
    mesh=_mesh,
    scratch_types=[
        pltpu.VMEM((ROWS_PER_W, IDX_COLS), jnp.int32),
        pltpu.VMEM((ROWS_PER_W, IDX_COLS), jnp.int32),
        pltpu.VMEM((ROWS_PER_W, IDX_COLS), jnp.int32),
        pltpu.VMEM((BPW, D), jnp.float32),
        pltpu.VMEM((BPW, D), jnp.float32),
        pltpu.VMEM((BPW, D), jnp.float32),
        pltpu.SemaphoreType.DMA,
        pltpu.SemaphoreType.DMA,
    ],
    compiler_params=pltpu.CompilerParams(use_tc_tiling_on_sc=False),
)


def _sc_quant_body(idx_h, cbt_h, q_o, idxv, rows, sem):
    wid = _wid()
    base = wid * BPW
    row0 = wid * ROWS_PER_W
    pltpu.sync_copy(idx_h.at[pl.ds(row0, ROWS_PER_W)], idxv)
    copies = []
    for j in range(NCH):
        copies.append(pltpu.async_copy(cbt_h.at[idxv.at[j]],
                                       rows.at[pl.ds(j * CH, CH)], sem))
    for c in copies:
        c.wait()
    pltpu.sync_copy(rows, q_o.at[pl.ds(base, BPW)])


_sc_quant = pl.kernel(
    _sc_quant_body,
    out_type=jax.ShapeDtypeStruct((B, D), jnp.float32),
    mesh=_mesh,
    scratch_types=[
        pltpu.VMEM((ROWS_PER_W, IDX_COLS), jnp.int32),
        pltpu.VMEM((BPW, D), jnp.float32),
        pltpu.SemaphoreType.DMA,
    ],
    compiler_params=pltpu.CompilerParams(use_tc_tiling_on_sc=False),
)

BS = 512


def _vq_body(x_ref, cb_ref, c2_ref, idx_ref, dsum_ref):
    x = x_ref[...]
    m = jnp.dot(2.0 * x, cb_ref[...], preferred_element_type=jnp.float32)
    x2 = jnp.sum(x * x, axis=1, keepdims=True)
    dist = (x2 - m) + c2_ref[...]
    rowmin = jnp.min(dist, axis=1, keepdims=True)
    eiota = lax.broadcasted_iota(jnp.int32, dist.shape, 1)
    idx = jnp.min(jnp.where(dist == rowmin, eiota, E), axis=1)
    idx_ref[...] = idx.astype(jnp.int32)

    @pl.when(pl.program_id(0) == 0)
    def _():
        dsum_ref[0, 0] = 0.0

    dsum_ref[0, 0] += jnp.sum(rowmin)


_vq = pl.pallas_call(
    _vq_body,
    grid=(B // BS,),
    in_specs=[
        pl.BlockSpec((BS, D), lambda i: (i, 0)),
        pl.BlockSpec((D, E), lambda i: (0, 0)),
        pl.BlockSpec((1, E), lambda i: (0, 0)),
    ],
    out_specs=[
        pl.BlockSpec((BS,), lambda i: (i,)),
        pl.BlockSpec((1, 1), lambda i: (0, 0), memory_space=pltpu.SMEM),
    ],
    out_shape=[
        jax.ShapeDtypeStruct((B,), jnp.int32),
        jax.ShapeDtypeStruct((1, 1), jnp.float32),
    ],
)


def kernel(user_id, interacted_items, pos, neg, item_table, user_table, codebook):
    del interacted_items
    uid2 = user_id.astype(jnp.int32).reshape(B // IDX_COLS, IDX_COLS)
    pos2 = pos.astype(jnp.int32).reshape(B // IDX_COLS, IDX_COLS)
    neg2 = neg.astype(jnp.int32).reshape(B // IDX_COLS, IDX_COLS)
    user_embed, pos_item, neg_item = _sc_gather3(
        uid2, pos2, neg2, item_table, user_table)
    c2 = jnp.sum(codebook ** 2, axis=0, keepdims=True)
    idx, dsum = _vq(user_embed, codebook, c2)
    quant = _sc_quant(idx.reshape(B // IDX_COLS, IDX_COLS), codebook.T)
    diff = (dsum[0, 0] / (B * D)).astype(jnp.float32)
    return (quant, pos_item, neg_item, diff, user_embed)
